# baseline (speedup 1.0000x reference)
import jax
import jax.numpy as jnp
from jax.experimental import pallas as pl
from jax.experimental.pallas import tpu as pltpu
from jax.experimental.pallas import tpu_sc as plsc

_NC = 19
_CM = 95
_IGNORE = -1
_PPC_W = 0.01
_PPD_W = 0.001
_UNCER_W = 1.0

_LN2 = 0.6931471805599453
_LOG2E = 1.4426950408889634

_STEPS = 16
_ROWS = 131072 // _STEPS
_SPB = _STEPS // 8
_HCHUNK = 128 // _SPB

_NW = 32
_RPW = 131072 // _NW
_SC_CH = 512
_SC_GRP = 16


def _tc_body(seg_ref, tgt_ref, conf_ref, x_ref, ct_ref, g_ref, bb_ref, out_ref):
    i = pl.program_id(0)

    seg = seg_ref[0]
    tgt = tgt_ref[0]
    conf = conf_ref[0]
    valid = tgt != _IGNORE
    vf = valid.astype(jnp.float32)
    tc = jnp.clip(tgt, 0, _NC - 1)

    m = jnp.max(seg, axis=0)
    cls_iota = jax.lax.broadcasted_iota(jnp.int32, seg.shape, 0)
    amax = jnp.min(jnp.where(seg == m[None], cls_iota, _NC), axis=0)
    seg2 = jnp.where(cls_iota == amax[None], -jnp.inf, seg)
    m2 = jnp.max(seg2, axis=0)

    s = jnp.sum(jnp.exp(seg - m[None]), axis=0)
    lse = m + jnp.log(s)
    seg_t = jnp.sum(jnp.where(cls_iota == tc[None], seg, 0.0), axis=0)
    nll_sum = jnp.sum((lse - seg_t) * vf)

    label = amax == tgt
    p = 1.0 / (1.0 + jnp.exp(m2 - m))
    u = jnp.where(label, 1.0 - p, p)
    bce = jnp.maximum(conf, 0.0) - conf * u + jnp.log1p(jnp.exp(-jnp.abs(conf)))
    bce_sum = jnp.sum(bce * vf)
    cnt = jnp.sum(vf)

    x = x_ref[...]
    ct = ct_ref[0, 0, :]

    ones_i = jnp.full((_CM, _CM), 1.0 / _CM, jnp.float32)
    ones_m = jnp.full((_CM, _CM), 1.0, jnp.float32)
    dn = (((1,), (0,)), ((), ()))
    mu = jax.lax.dot_general(x, ones_i, dn,
                             preferred_element_type=jnp.float32)
    ex2 = jax.lax.dot_general(x * x, ones_i, dn,
                              preferred_element_type=jnp.float32)
    v2 = (ex2 - mu * mu) * (_LN2 * _LN2) + (1e-5 * _LN2 * _LN2)
    rs2 = jax.lax.rsqrt(v2)
    n2 = (x - mu) * rs2 * g_ref[0][None, :] + bb_ref[0][None, :]
    es = jnp.exp2(n2)
    s3 = jax.lax.dot_general(es, ones_m, dn,
                             preferred_element_type=jnp.float32)
    l2s3 = jnp.log2(s3)

    lane = jax.lax.broadcasted_iota(jnp.int32, (_ROWS, _CM), 1)
    oh = lane == ct[:, None]
    combo = jnp.sum(jnp.where(oh, l2s3 - n2, 0.0))

    @pl.when(i == 0)
    def _():
        out_ref[0] = 0.0
        out_ref[1] = 0.0
        out_ref[2] = 0.0
        out_ref[3] = 0.0

    out_ref[0] += nll_sum
    out_ref[1] += bce_sum
    out_ref[2] += cnt
    out_ref[3] += combo


def _sc_body(x_hbm, ct_hbm, out_hbm, xbuf, ctbuf, pbuf, cbuf):
    c = jax.lax.axis_index("c")
    s = jax.lax.axis_index("s")
    wid = s * 2 + c
    base = wid * _RPW

    accp = jnp.float32(0.0)
    accc = jnp.zeros((_SC_GRP,), jnp.float32)
    for ch in range(_RPW // _SC_CH):
        rb = base + ch * _SC_CH
        pltpu.sync_copy(x_hbm.at[pl.ds(rb, _SC_CH), :], xbuf)
        pltpu.sync_copy(ct_hbm.at[pl.ds(rb, _SC_CH)], ctbuf)

        def grp(g, carry):
            ap, ac = carry
            gb = g * _SC_GRP
            ct16 = ctbuf[pl.ds(gb, _SC_GRP)]
            ac = ac + jnp.where(ct16 != _IGNORE, 1.0, 0.0)
            for k in range(_SC_GRP):
                ctr = ct16[k]
                cc = jnp.clip(ctr, 0, _CM - 1)
                v = xbuf[gb + k, pl.ds(cc, _SC_GRP)]
                d = 1.0 - v[0]
                ap = ap + jnp.where(ctr != _IGNORE, d * d, 0.0)
            return ap, ac

        accp, accc = jax.lax.fori_loop(0, _SC_CH // _SC_GRP, grp, (accp, accc))

    lane16 = jax.lax.iota(jnp.int32, _SC_GRP)
    pbuf[...] = jnp.where(lane16 == 0, jnp.broadcast_to(accp, (_SC_GRP,)), 0.0)
    cbuf[...] = accc
    pltpu.sync_copy(pbuf, out_hbm.at[wid, 0])
    pltpu.sync_copy(cbuf, out_hbm.at[wid, 1])


def kernel(seg, confidence, contrast_logits, contrast_target, target, ln_gamma, ln_beta):
    n = contrast_target.shape[0]
    ct3 = contrast_target.reshape(_STEPS, 1, n // _STEPS)
    g2 = ln_gamma.reshape(1, _CM)
    bb2 = (ln_beta * _LOG2E).reshape(1, _CM)

    sc_parts = pl.kernel(
        _sc_body,
        out_type=jax.ShapeDtypeStruct((_NW, 2, _SC_GRP), jnp.float32),
        mesh=plsc.VectorSubcoreMesh(core_axis_name="c", subcore_axis_name="s"),
        scratch_types=[
            pltpu.VMEM((_SC_CH, _CM), jnp.float32),
            pltpu.VMEM((_SC_CH,), jnp.int32),
            pltpu.VMEM((_SC_GRP,), jnp.float32),
            pltpu.VMEM((_SC_GRP,), jnp.float32),
        ],
    )(contrast_logits, contrast_target)

    sums = pl.pallas_call(
        _tc_body,
        grid=(_STEPS,),
        in_specs=[
            pl.BlockSpec((1, _NC, _HCHUNK, 128), lambda i: (i // _SPB, 0, i % _SPB, 0)),
            pl.BlockSpec((1, _HCHUNK, 128), lambda i: (i // _SPB, i % _SPB, 0)),
            pl.BlockSpec((1, _HCHUNK, 128), lambda i: (i // _SPB, i % _SPB, 0)),
            pl.BlockSpec((_ROWS, _CM), lambda i: (i, 0)),
            pl.BlockSpec((1, 1, _ROWS), lambda i: (i, 0, 0)),
            pl.BlockSpec((1, _CM), lambda i: (0, 0)),
            pl.BlockSpec((1, _CM), lambda i: (0, 0)),
        ],
        out_specs=pl.BlockSpec(memory_space=pltpu.SMEM),
        out_shape=jax.ShapeDtypeStruct((4,), jnp.float32),
    )(seg, target, confidence, contrast_logits, ct3, g2, bb2)

    nll_sum, bce_sum, cnt, combo = sums[0], sums[1], sums[2], sums[3]
    ppd_raw = jnp.sum(sc_parts[:, 0, :])
    ccnt = jnp.sum(sc_parts[:, 1, :])

    seg_loss = nll_sum / jnp.maximum(cnt, 1.0)
    uncer = bce_sum / jnp.maximum(cnt, 1.0)
    contrast = (_PPC_W * _LN2 * combo + _PPD_W * ppd_raw) / jnp.maximum(ccnt, 1.0)
    return seg_loss + contrast + _UNCER_W * uncer
